# Initial kernel scaffold; baseline (speedup 1.0000x reference)
import functools

import jax
import jax.numpy as jnp
from jax import lax
from jax.experimental import pallas as pl
from jax.experimental.pallas import tpu as pltpu
from jax.experimental.pallas import tpu_sc as plsc

N = 10000
E = 320000
D = 128
G = 64
OUT = 16
VPAD = 512

NC = 2
NS = 16
NW = NC * NS
EPW = E // NW
CH = 80
NCHUNK = EPW // CH
RPT = N // NS
ZR = 125


def _edge_agg_partials(feats, src3, dst3):
    mesh = plsc.VectorSubcoreMesh(core_axis_name="c", subcore_axis_name="s")

    @functools.partial(
        pl.kernel,
        out_type=jax.ShapeDtypeStruct((NC, N, D), jnp.float32),
        mesh=mesh,
        scratch_types=[
            pltpu.VMEM((NCHUNK, CH), jnp.int32),
            pltpu.VMEM((NCHUNK, CH), jnp.int32),
            pltpu.VMEM((CH, D), jnp.float32),
            pltpu.VMEM((ZR, D), jnp.float32),
            pltpu.VMEM_SHARED((N, D), jnp.float32),
        ],
    )
    def k(feats_hbm, src_hbm, dst_hbm, out_hbm, srcv, dstv, rows, zbuf, acc):
        cid = lax.axis_index("c")
        sid = lax.axis_index("s")
        wid = cid * NS + sid

        @pl.loop(0, ZR)
        def _(i):
            @pl.loop(0, D, step=16)
            def _(j):
                zbuf.at[i, pl.ds(j, 16)][...] = jnp.zeros((16,), jnp.float32)

        @pl.loop(0, RPT, step=ZR)
        def _(r):
            pltpu.sync_copy(zbuf, acc.at[pl.ds(sid * RPT + r, ZR)])

        plsc.subcore_barrier()

        pltpu.sync_copy(src_hbm.at[wid], srcv)
        pltpu.sync_copy(dst_hbm.at[wid], dstv)

        @pl.loop(0, NCHUNK)
        def _(j):
            pltpu.sync_copy(feats_hbm.at[srcv.at[j]], rows)
            pltpu.sync_copy(rows, acc.at[dstv.at[j]], add=True)

        plsc.subcore_barrier()

        @pl.loop(0, RPT, step=ZR)
        def _(r):
            pltpu.sync_copy(
                acc.at[pl.ds(sid * RPT + r, ZR)],
                out_hbm.at[cid, pl.ds(sid * RPT + r, ZR)],
            )

    return k(feats, src3, dst3)


def _embed_kernel(x_ref, emb_ref, out_ref):
    xb = x_ref[0]
    ids = jax.lax.broadcasted_iota(jnp.int32, (VPAD, xb.shape[1]), 0)
    onehot = (ids == xb).astype(jnp.float32)
    out_ref[...] = jax.lax.dot_general(
        onehot, emb_ref[...], (((0,), (0,)), ((), ())),
        preferred_element_type=jnp.float32)


def _embed(x, emb_pad):
    blk = 1000
    x3 = x.reshape(N // blk, 1, blk)
    return pl.pallas_call(
        _embed_kernel,
        out_shape=jax.ShapeDtypeStruct((N, D), jnp.float32),
        grid=(N // blk,),
        in_specs=[
            pl.BlockSpec((1, 1, blk), lambda i: (i, 0, 0)),
            pl.BlockSpec((VPAD, D), lambda i: (0, 0)),
        ],
        out_specs=pl.BlockSpec((blk, D), lambda i: (i, 0)),
    )(x3, emb_pad)


def _mlp_kernel(x_ref, p_ref, Wa_ref, ba_ref, g_ref, be_ref, Wb_ref, bb_ref,
                out_ref):
    hin = x_ref[...] + p_ref[0] + p_ref[1]
    t = jnp.dot(hin, Wa_ref[...], preferred_element_type=jnp.float32)
    t = t + ba_ref[...]
    mu = jnp.mean(t, axis=0, keepdims=True)
    var = jnp.mean(jnp.square(t - mu), axis=0, keepdims=True)
    tn = g_ref[...] * (t - mu) * jax.lax.rsqrt(var + 1e-5) + be_ref[...]
    r = jnp.maximum(tn, 0.0)
    h = jnp.dot(r, Wb_ref[...], preferred_element_type=jnp.float32)
    out_ref[...] = jnp.maximum(h + bb_ref[...], 0.0)


def _gin_mlp(xin, partials, Wa, ba, g, be, Wb, bb):
    return pl.pallas_call(
        _mlp_kernel,
        out_shape=jax.ShapeDtypeStruct((N, D), jnp.float32),
    )(xin, partials, Wa, ba.reshape(1, D), g.reshape(1, D), be.reshape(1, D),
      Wb, bb.reshape(1, D))


def _pool_kernel(h1_ref, h2_ref, b_ref, W1_ref, b1_ref, W2_ref, b2_ref,
                 out_ref):
    gids = jax.lax.broadcasted_iota(jnp.int32, (G, N), 0)
    onehot = (gids == b_ref[...]).astype(jnp.float32)
    hg1 = jnp.dot(onehot, h1_ref[...], preferred_element_type=jnp.float32)
    hg2 = jnp.dot(onehot, h2_ref[...], preferred_element_type=jnp.float32)
    hg = jnp.concatenate([hg1, hg2], axis=1)
    y = jnp.dot(hg, W1_ref[...], preferred_element_type=jnp.float32)
    y = jnp.maximum(y + b1_ref[...], 0.0)
    out_ref[...] = jnp.dot(y, W2_ref[...],
                           preferred_element_type=jnp.float32) + b2_ref[...]


def _pool_head(h1, h2, batch, lin1_W, lin1_b, lin2_W, lin2_b):
    return pl.pallas_call(
        _pool_kernel,
        out_shape=jax.ShapeDtypeStruct((G, OUT), jnp.float32),
    )(h1, h2, batch.reshape(1, N), lin1_W, lin1_b.reshape(1, D), lin2_W,
      lin2_b.reshape(1, OUT))


def kernel(x, edge_index, batch, emb, W1a, b1a, g1, be1, W1b, b1b,
           W2a, b2a, g2, be2, W2b, b2b, lin1_W, lin1_b, lin2_W, lin2_b):
    emb_pad = jnp.zeros((VPAD, D), jnp.float32).at[:emb.shape[0]].set(emb)
    src3 = edge_index[0].reshape(NW, NCHUNK, CH)
    dst3 = edge_index[1].reshape(NW, NCHUNK, CH)

    feats = _embed(x.astype(jnp.int32), emb_pad)

    p1 = _edge_agg_partials(feats, src3, dst3)
    h1 = _gin_mlp(feats, p1, W1a, b1a, g1, be1, W1b, b1b)

    p2 = _edge_agg_partials(h1, src3, dst3)
    h2 = _gin_mlp(h1, p2, W2a, b2a, g2, be2, W2b, b2b)

    out = _pool_head(h1, h2, batch, lin1_W, lin1_b, lin2_W, lin2_b)
    return (out, feats)


# speedup vs baseline: 7.3571x; 7.3571x over previous
import functools

import jax
import jax.numpy as jnp
from jax import lax
from jax.experimental import pallas as pl
from jax.experimental.pallas import tpu as pltpu
from jax.experimental.pallas import tpu_sc as plsc

N = 10000
E = 320000
D = 128
G = 64
OUT = 16
VPAD = 512

NC = 2
NS = 16
NW = NC * NS
EPW = E // NW
CH = 80
NCHUNK = EPW // CH
NP = 10240
RPT = NP // NS


def _edge_agg_partials(feats, src3, dst3):
    mesh = plsc.VectorSubcoreMesh(core_axis_name="c", subcore_axis_name="s")

    @functools.partial(
        pl.kernel,
        out_type=jax.ShapeDtypeStruct((NC, NP, D), jnp.float32),
        mesh=mesh,
        scratch_types=[
            pltpu.VMEM((NCHUNK, CH), jnp.int32),
            pltpu.VMEM((NCHUNK, CH), jnp.int32),
            pltpu.VMEM((CH, D), jnp.float32),
            pltpu.VMEM_SHARED((NP, D), jnp.float32),
        ],
    )
    def k(feats_hbm, src_hbm, dst_hbm, out_hbm, srcv, dstv, rows, acc):
        cid = lax.axis_index("c")
        sid = lax.axis_index("s")
        wid = cid * NS + sid

        @pl.loop(0, CH)
        def _(i):
            @pl.loop(0, D, step=16)
            def _(j):
                rows.at[i, pl.ds(j, 16)][...] = jnp.zeros((16,), jnp.float32)

        @pl.loop(0, RPT, step=CH)
        def _(r):
            pltpu.sync_copy(rows, acc.at[pl.ds(sid * RPT + r, CH)])

        plsc.subcore_barrier()

        pltpu.sync_copy(src_hbm.at[wid], srcv)
        pltpu.sync_copy(dst_hbm.at[wid], dstv)

        @pl.loop(0, NCHUNK)
        def _(j):
            pltpu.sync_copy(feats_hbm.at[srcv.at[j]], rows)
            pltpu.sync_copy(rows, acc.at[dstv.at[j]], add=True)

        plsc.subcore_barrier()

        @pl.loop(0, RPT, step=CH)
        def _(r):
            pltpu.sync_copy(
                acc.at[pl.ds(sid * RPT + r, CH)],
                out_hbm.at[cid, pl.ds(sid * RPT + r, CH)],
            )

    return k(feats, src3, dst3)


def _embed_kernel(x_ref, emb_ref, out_ref):
    xb = x_ref[0]
    ids = jax.lax.broadcasted_iota(jnp.int32, (VPAD, xb.shape[1]), 0)
    onehot = (ids == xb).astype(jnp.float32)
    out_ref[...] = jax.lax.dot_general(
        onehot, emb_ref[...], (((0,), (0,)), ((), ())),
        preferred_element_type=jnp.float32)


def _embed(x, emb_pad):
    blk = 1000
    x3 = x.reshape(N // blk, 1, blk)
    return pl.pallas_call(
        _embed_kernel,
        out_shape=jax.ShapeDtypeStruct((N, D), jnp.float32),
        grid=(N // blk,),
        in_specs=[
            pl.BlockSpec((1, 1, blk), lambda i: (i, 0, 0)),
            pl.BlockSpec((VPAD, D), lambda i: (0, 0)),
        ],
        out_specs=pl.BlockSpec((blk, D), lambda i: (i, 0)),
    )(x3, emb_pad)


def _mlp_kernel(x_ref, p_ref, Wa_ref, ba_ref, g_ref, be_ref, Wb_ref, bb_ref,
                out_ref):
    hin = x_ref[...] + p_ref[0, :N] + p_ref[1, :N]
    t = jnp.dot(hin, Wa_ref[...], preferred_element_type=jnp.float32)
    t = t + ba_ref[...]
    mu = jnp.mean(t, axis=0, keepdims=True)
    var = jnp.mean(jnp.square(t - mu), axis=0, keepdims=True)
    tn = g_ref[...] * (t - mu) * jax.lax.rsqrt(var + 1e-5) + be_ref[...]
    r = jnp.maximum(tn, 0.0)
    h = jnp.dot(r, Wb_ref[...], preferred_element_type=jnp.float32)
    out_ref[...] = jnp.maximum(h + bb_ref[...], 0.0)


def _gin_mlp(xin, partials, Wa, ba, g, be, Wb, bb):
    return pl.pallas_call(
        _mlp_kernel,
        out_shape=jax.ShapeDtypeStruct((N, D), jnp.float32),
    )(xin, partials, Wa, ba.reshape(1, D), g.reshape(1, D), be.reshape(1, D),
      Wb, bb.reshape(1, D))


def _pool_kernel(h1_ref, h2_ref, b_ref, W1_ref, b1_ref, W2_ref, b2_ref,
                 out_ref):
    gids = jax.lax.broadcasted_iota(jnp.int32, (G, N), 0)
    onehot = (gids == b_ref[...]).astype(jnp.float32)
    hg1 = jnp.dot(onehot, h1_ref[...], preferred_element_type=jnp.float32)
    hg2 = jnp.dot(onehot, h2_ref[...], preferred_element_type=jnp.float32)
    hg = jnp.concatenate([hg1, hg2], axis=1)
    y = jnp.dot(hg, W1_ref[...], preferred_element_type=jnp.float32)
    y = jnp.maximum(y + b1_ref[...], 0.0)
    out_ref[...] = jnp.dot(y, W2_ref[...],
                           preferred_element_type=jnp.float32) + b2_ref[...]


def _pool_head(h1, h2, batch, lin1_W, lin1_b, lin2_W, lin2_b):
    return pl.pallas_call(
        _pool_kernel,
        out_shape=jax.ShapeDtypeStruct((G, OUT), jnp.float32),
    )(h1, h2, batch.reshape(1, N), lin1_W, lin1_b.reshape(1, D), lin2_W,
      lin2_b.reshape(1, OUT))


def kernel(x, edge_index, batch, emb, W1a, b1a, g1, be1, W1b, b1b,
           W2a, b2a, g2, be2, W2b, b2b, lin1_W, lin1_b, lin2_W, lin2_b):
    emb_pad = jnp.zeros((VPAD, D), jnp.float32).at[:emb.shape[0]].set(emb)
    src3 = edge_index[0].reshape(NW, NCHUNK, CH)
    dst3 = edge_index[1].reshape(NW, NCHUNK, CH)

    feats = _embed(x.astype(jnp.int32), emb_pad)

    p1 = _edge_agg_partials(feats, src3, dst3)
    h1 = _gin_mlp(feats, p1, W1a, b1a, g1, be1, W1b, b1b)

    p2 = _edge_agg_partials(h1, src3, dst3)
    h2 = _gin_mlp(h1, p2, W2a, b2a, g2, be2, W2b, b2b)

    out = _pool_head(h1, h2, batch, lin1_W, lin1_b, lin2_W, lin2_b)
    return (out, feats)


# speedup vs baseline: 9.8587x; 1.3400x over previous
import functools

import jax
import jax.numpy as jnp
from jax import lax
from jax.experimental import pallas as pl
from jax.experimental.pallas import tpu as pltpu
from jax.experimental.pallas import tpu_sc as plsc

N = 10000
E = 320000
D = 128
G = 64
OUT = 16
VPAD = 512

NC = 2
NS = 16
NW = NC * NS
EPW = E // NW
CH = 128
NCHUNK = 80
EPWP = NCHUNK * CH
PADE = EPWP - EPW
NB = 2
IW = 16
NWIN = NCHUNK // IW
NP = 10240
RPT = NP // NS


def _edge_agg_partials(feats, src3, dst3):
    mesh = plsc.VectorSubcoreMesh(core_axis_name="c", subcore_axis_name="s")

    @functools.partial(
        pl.kernel,
        out_type=jax.ShapeDtypeStruct((NC, NP, D), jnp.float32),
        mesh=mesh,
        scratch_types=[
            pltpu.VMEM((2, IW, CH), jnp.int32),
            pltpu.VMEM((2, IW, CH), jnp.int32),
            pltpu.VMEM((NB, CH, D), jnp.float32),
            pltpu.VMEM_SHARED((NP, D), jnp.float32),
            pltpu.SemaphoreType.DMA,
            pltpu.SemaphoreType.DMA,
        ] + [pltpu.SemaphoreType.DMA] * (2 * NB),
    )
    def k(feats_hbm, src_hbm, dst_hbm, out_hbm, srcw, dstw, rows, acc,
          isem0, isem1, *sems):
        isem = (isem0, isem1)
        gsem = sems[:NB]
        ssem = sems[NB:]
        cid = lax.axis_index("c")
        sid = lax.axis_index("s")
        wid = cid * NS + sid

        def idx_fetch(w):
            p = w % 2
            pltpu.async_copy(src_hbm.at[wid, pl.ds(w * IW, IW)], srcw.at[p],
                             isem[p])
            pltpu.async_copy(dst_hbm.at[wid, pl.ds(w * IW, IW)], dstw.at[p],
                             isem[p])

        def idx_wait(w):
            p = w % 2
            pltpu.make_async_copy(src_hbm.at[wid, pl.ds(w * IW, IW)],
                                  srcw.at[p], isem[p]).wait()
            pltpu.make_async_copy(dst_hbm.at[wid, pl.ds(w * IW, IW)],
                                  dstw.at[p], isem[p]).wait()

        idx_fetch(0)

        zrow = rows.at[0]
        @pl.loop(0, CH)
        def _(i):
            @pl.loop(0, D, step=16)
            def _(j):
                zrow.at[i, pl.ds(j, 16)][...] = jnp.zeros((16,), jnp.float32)

        @pl.loop(0, RPT, step=CH)
        def _(r):
            pltpu.sync_copy(zrow, acc.at[pl.ds(sid * RPT + r, CH)])

        plsc.subcore_barrier()

        def gather(p, j, b):
            pltpu.async_copy(feats_hbm.at[srcw.at[p, j]], rows.at[b], gsem[b])

        def gather_wait(p, j, b):
            pltpu.make_async_copy(
                feats_hbm.at[srcw.at[p, j]], rows.at[b], gsem[b]).wait()

        def scat(p, j, b):
            pltpu.async_copy(rows.at[b], acc.at[dstw.at[p, j]], ssem[b],
                             add=True)

        def scat_wait(p, j, b):
            pltpu.make_async_copy(
                rows.at[b], acc.at[dstw.at[p, j]], ssem[b]).wait()

        for w in range(NWIN):
            p = w % 2
            idx_wait(w)
            if w + 1 < NWIN:
                idx_fetch(w + 1)
            for b in range(NB):
                gather(p, b, b)

            @pl.loop(0, IW // NB - 1)
            def _(g):
                base = g * NB
                for b in range(NB):
                    gather_wait(p, base + b, b)
                    scat(p, base + b, b)
                for b in range(NB):
                    scat_wait(p, base + b, b)
                    gather(p, base + NB + b, b)

            last = IW - NB
            for b in range(NB):
                gather_wait(p, last + b, b)
                scat(p, last + b, b)
            for b in range(NB):
                scat_wait(p, last + b, b)

        plsc.subcore_barrier()

        @pl.loop(0, RPT, step=CH)
        def _(r):
            pltpu.sync_copy(
                acc.at[pl.ds(sid * RPT + r, CH)],
                out_hbm.at[cid, pl.ds(sid * RPT + r, CH)],
            )

    return k(feats, src3, dst3)


def _embed_kernel(x_ref, emb_ref, out_ref):
    xb = x_ref[0]
    ids = jax.lax.broadcasted_iota(jnp.int32, (VPAD, xb.shape[1]), 0)
    onehot = (ids == xb).astype(jnp.float32)
    out_ref[...] = jax.lax.dot_general(
        onehot, emb_ref[...], (((0,), (0,)), ((), ())),
        preferred_element_type=jnp.float32)


def _embed(x, emb_pad):
    blk = 1000
    x3 = x.reshape(N // blk, 1, blk)
    return pl.pallas_call(
        _embed_kernel,
        out_shape=jax.ShapeDtypeStruct((N, D), jnp.float32),
        grid=(N // blk,),
        in_specs=[
            pl.BlockSpec((1, 1, blk), lambda i: (i, 0, 0)),
            pl.BlockSpec((VPAD, D), lambda i: (0, 0)),
        ],
        out_specs=pl.BlockSpec((blk, D), lambda i: (i, 0)),
    )(x3, emb_pad)


def _mlp_kernel(x_ref, p_ref, Wa_ref, ba_ref, g_ref, be_ref, Wb_ref, bb_ref,
                out_ref):
    hin = x_ref[...] + p_ref[0, :N] + p_ref[1, :N]
    t = jnp.dot(hin, Wa_ref[...], preferred_element_type=jnp.float32)
    t = t + ba_ref[...]
    mu = jnp.mean(t, axis=0, keepdims=True)
    var = jnp.mean(jnp.square(t - mu), axis=0, keepdims=True)
    tn = g_ref[...] * (t - mu) * jax.lax.rsqrt(var + 1e-5) + be_ref[...]
    r = jnp.maximum(tn, 0.0)
    h = jnp.dot(r, Wb_ref[...], preferred_element_type=jnp.float32)
    out_ref[...] = jnp.maximum(h + bb_ref[...], 0.0)


def _gin_mlp(xin, partials, Wa, ba, g, be, Wb, bb):
    return pl.pallas_call(
        _mlp_kernel,
        out_shape=jax.ShapeDtypeStruct((N, D), jnp.float32),
    )(xin, partials, Wa, ba.reshape(1, D), g.reshape(1, D), be.reshape(1, D),
      Wb, bb.reshape(1, D))


def _pool_kernel(h1_ref, h2_ref, b_ref, W1_ref, b1_ref, W2_ref, b2_ref,
                 out_ref):
    gids = jax.lax.broadcasted_iota(jnp.int32, (G, N), 0)
    onehot = (gids == b_ref[...]).astype(jnp.float32)
    hg1 = jnp.dot(onehot, h1_ref[...], preferred_element_type=jnp.float32)
    hg2 = jnp.dot(onehot, h2_ref[...], preferred_element_type=jnp.float32)
    hg = jnp.concatenate([hg1, hg2], axis=1)
    y = jnp.dot(hg, W1_ref[...], preferred_element_type=jnp.float32)
    y = jnp.maximum(y + b1_ref[...], 0.0)
    out_ref[...] = jnp.dot(y, W2_ref[...],
                           preferred_element_type=jnp.float32) + b2_ref[...]


def _pool_head(h1, h2, batch, lin1_W, lin1_b, lin2_W, lin2_b):
    return pl.pallas_call(
        _pool_kernel,
        out_shape=jax.ShapeDtypeStruct((G, OUT), jnp.float32),
    )(h1, h2, batch.reshape(1, N), lin1_W, lin1_b.reshape(1, D), lin2_W,
      lin2_b.reshape(1, OUT))


def kernel(x, edge_index, batch, emb, W1a, b1a, g1, be1, W1b, b1b,
           W2a, b2a, g2, be2, W2b, b2b, lin1_W, lin1_b, lin2_W, lin2_b):
    emb_pad = jnp.zeros((VPAD, D), jnp.float32).at[:emb.shape[0]].set(emb)
    pad_src = jnp.broadcast_to((37 * jnp.arange(PADE, dtype=jnp.int32)) % N,
                               (NW, PADE))
    pad_dst = jnp.broadcast_to(N + jnp.arange(PADE, dtype=jnp.int32),
                               (NW, PADE))
    src3 = jnp.concatenate(
        [edge_index[0].reshape(NW, EPW), pad_src], axis=1
    ).reshape(NW, NCHUNK, CH)
    dst3 = jnp.concatenate(
        [edge_index[1].reshape(NW, EPW), pad_dst], axis=1
    ).reshape(NW, NCHUNK, CH)

    feats = _embed(x.astype(jnp.int32), emb_pad)

    p1 = _edge_agg_partials(feats, src3, dst3)
    h1 = _gin_mlp(feats, p1, W1a, b1a, g1, be1, W1b, b1b)

    p2 = _edge_agg_partials(h1, src3, dst3)
    h2 = _gin_mlp(h1, p2, W2a, b2a, g2, be2, W2b, b2b)

    out = _pool_head(h1, h2, batch, lin1_W, lin1_b, lin2_W, lin2_b)
    return (out, feats)


# speedup vs baseline: 11.7582x; 1.1927x over previous
import functools

import jax
import jax.numpy as jnp
from jax import lax
from jax.experimental import pallas as pl
from jax.experimental.pallas import tpu as pltpu
from jax.experimental.pallas import tpu_sc as plsc

N = 10000
E = 320000
D = 128
G = 64
OUT = 16
VPAD = 512

NC = 2
NS = 16
NW = NC * NS
EPW = E // NW
CH = 128
NCHUNK = 80
EPWP = NCHUNK * CH
PADE = EPWP - EPW
NB = 2
IW = 16
NWIN = NCHUNK // IW
NP = 10240
RPT = NP // NS


def _edge_agg_partials(feats, src3, dst3):
    mesh = plsc.VectorSubcoreMesh(core_axis_name="c", subcore_axis_name="s")

    @functools.partial(
        pl.kernel,
        out_type=jax.ShapeDtypeStruct((NC, NP, D), jnp.float32),
        mesh=mesh,
        scratch_types=[
            pltpu.VMEM((2, IW, CH), jnp.int32),
            pltpu.VMEM((2, IW, CH), jnp.int32),
            pltpu.VMEM((2 * NB, CH // 2, D), jnp.float32),
            pltpu.VMEM_SHARED((NP, D), jnp.float32),
            pltpu.SemaphoreType.DMA,
            pltpu.SemaphoreType.DMA,
        ] + [pltpu.SemaphoreType.DMA] * (4 * NB),
    )
    def k(feats_hbm, src_hbm, dst_hbm, out_hbm, srcw, dstw, rows, acc,
          isem0, isem1, *sems):
        isem = (isem0, isem1)
        gsem = sems[:2 * NB]
        ssem = sems[2 * NB:]
        cid = lax.axis_index("c")
        sid = lax.axis_index("s")
        wid = cid * NS + sid

        def idx_fetch(w):
            p = w % 2
            pltpu.async_copy(src_hbm.at[wid, pl.ds(w * IW, IW)], srcw.at[p],
                             isem[p])
            pltpu.async_copy(dst_hbm.at[wid, pl.ds(w * IW, IW)], dstw.at[p],
                             isem[p])

        def idx_wait(w):
            p = w % 2
            pltpu.make_async_copy(src_hbm.at[wid, pl.ds(w * IW, IW)],
                                  srcw.at[p], isem[p]).wait()
            pltpu.make_async_copy(dst_hbm.at[wid, pl.ds(w * IW, IW)],
                                  dstw.at[p], isem[p]).wait()

        idx_fetch(0)

        zrow = rows.at[0]
        @pl.loop(0, CH // 2)
        def _(i):
            @pl.loop(0, D, step=16)
            def _(j):
                zrow.at[i, pl.ds(j, 16)][...] = jnp.zeros((16,), jnp.float32)

        @pl.loop(0, RPT, step=CH // 2)
        def _(r):
            pltpu.sync_copy(zrow, acc.at[pl.ds(sid * RPT + r, CH // 2)])

        plsc.subcore_barrier()

        HC = CH // 2
        NSLOT = 2 * NB

        def subidx(arr, p, sj):
            return arr.at[p, sj // 2, pl.ds((sj % 2) * HC, HC)]

        def gather(p, sj, b):
            pltpu.async_copy(feats_hbm.at[subidx(srcw, p, sj)], rows.at[b],
                             gsem[b])

        def gather_wait(p, sj, b):
            pltpu.make_async_copy(
                feats_hbm.at[subidx(srcw, p, sj)], rows.at[b], gsem[b]).wait()

        def scat(p, sj, b):
            pltpu.async_copy(rows.at[b], acc.at[subidx(dstw, p, sj)], ssem[b],
                             add=True)

        def scat_wait(p, sj, b):
            pltpu.make_async_copy(
                rows.at[b], acc.at[subidx(dstw, p, sj)], ssem[b]).wait()

        NSJ = 2 * IW
        for w in range(NWIN):
            p = w % 2
            idx_wait(w)
            if w + 1 < NWIN:
                idx_fetch(w + 1)
            for b in range(NSLOT):
                gather(p, b, b)

            @pl.loop(0, NSJ // NSLOT - 1)
            def _(g):
                base = g * NSLOT
                for b in range(NSLOT):
                    gather_wait(p, base + b, b)
                    scat(p, base + b, b)
                for b in range(NSLOT):
                    scat_wait(p, base + b, b)
                    gather(p, base + NSLOT + b, b)

            last = NSJ - NSLOT
            for b in range(NSLOT):
                gather_wait(p, last + b, b)
                scat(p, last + b, b)
            for b in range(NSLOT):
                scat_wait(p, last + b, b)

        plsc.subcore_barrier()

        @pl.loop(0, RPT, step=CH)
        def _(r):
            pltpu.sync_copy(
                acc.at[pl.ds(sid * RPT + r, CH)],
                out_hbm.at[cid, pl.ds(sid * RPT + r, CH)],
            )

    return k(feats, src3, dst3)


def _embed_kernel(x_ref, emb_ref, out_ref):
    xb = x_ref[0]
    ids = jax.lax.broadcasted_iota(jnp.int32, (VPAD, xb.shape[1]), 0)
    onehot = (ids == xb).astype(jnp.float32)
    out_ref[...] = jax.lax.dot_general(
        onehot, emb_ref[...], (((0,), (0,)), ((), ())),
        preferred_element_type=jnp.float32)


def _embed(x, emb_pad):
    blk = 1000
    x3 = x.reshape(N // blk, 1, blk)
    return pl.pallas_call(
        _embed_kernel,
        out_shape=jax.ShapeDtypeStruct((N, D), jnp.float32),
        grid=(N // blk,),
        in_specs=[
            pl.BlockSpec((1, 1, blk), lambda i: (i, 0, 0)),
            pl.BlockSpec((VPAD, D), lambda i: (0, 0)),
        ],
        out_specs=pl.BlockSpec((blk, D), lambda i: (i, 0)),
    )(x3, emb_pad)


def _mlp_kernel(x_ref, p_ref, Wa_ref, ba_ref, g_ref, be_ref, Wb_ref, bb_ref,
                out_ref):
    hin = x_ref[...] + p_ref[0, :N] + p_ref[1, :N]
    t = jnp.dot(hin, Wa_ref[...], preferred_element_type=jnp.float32)
    t = t + ba_ref[...]
    mu = jnp.mean(t, axis=0, keepdims=True)
    var = jnp.mean(jnp.square(t - mu), axis=0, keepdims=True)
    tn = g_ref[...] * (t - mu) * jax.lax.rsqrt(var + 1e-5) + be_ref[...]
    r = jnp.maximum(tn, 0.0)
    h = jnp.dot(r, Wb_ref[...], preferred_element_type=jnp.float32)
    out_ref[...] = jnp.maximum(h + bb_ref[...], 0.0)


def _gin_mlp(xin, partials, Wa, ba, g, be, Wb, bb):
    return pl.pallas_call(
        _mlp_kernel,
        out_shape=jax.ShapeDtypeStruct((N, D), jnp.float32),
    )(xin, partials, Wa, ba.reshape(1, D), g.reshape(1, D), be.reshape(1, D),
      Wb, bb.reshape(1, D))


def _pool_kernel(h1_ref, h2_ref, b_ref, W1_ref, b1_ref, W2_ref, b2_ref,
                 out_ref):
    gids = jax.lax.broadcasted_iota(jnp.int32, (G, N), 0)
    onehot = (gids == b_ref[...]).astype(jnp.float32)
    hg1 = jnp.dot(onehot, h1_ref[...], preferred_element_type=jnp.float32)
    hg2 = jnp.dot(onehot, h2_ref[...], preferred_element_type=jnp.float32)
    hg = jnp.concatenate([hg1, hg2], axis=1)
    y = jnp.dot(hg, W1_ref[...], preferred_element_type=jnp.float32)
    y = jnp.maximum(y + b1_ref[...], 0.0)
    out_ref[...] = jnp.dot(y, W2_ref[...],
                           preferred_element_type=jnp.float32) + b2_ref[...]


def _pool_head(h1, h2, batch, lin1_W, lin1_b, lin2_W, lin2_b):
    return pl.pallas_call(
        _pool_kernel,
        out_shape=jax.ShapeDtypeStruct((G, OUT), jnp.float32),
    )(h1, h2, batch.reshape(1, N), lin1_W, lin1_b.reshape(1, D), lin2_W,
      lin2_b.reshape(1, OUT))


def kernel(x, edge_index, batch, emb, W1a, b1a, g1, be1, W1b, b1b,
           W2a, b2a, g2, be2, W2b, b2b, lin1_W, lin1_b, lin2_W, lin2_b):
    emb_pad = jnp.zeros((VPAD, D), jnp.float32).at[:emb.shape[0]].set(emb)
    pad_src = jnp.broadcast_to((37 * jnp.arange(PADE, dtype=jnp.int32)) % N,
                               (NW, PADE))
    pad_dst = jnp.broadcast_to(N + jnp.arange(PADE, dtype=jnp.int32),
                               (NW, PADE))
    src3 = jnp.concatenate(
        [edge_index[0].reshape(NW, EPW), pad_src], axis=1
    ).reshape(NW, NCHUNK, CH)
    dst3 = jnp.concatenate(
        [edge_index[1].reshape(NW, EPW), pad_dst], axis=1
    ).reshape(NW, NCHUNK, CH)

    feats = _embed(x.astype(jnp.int32), emb_pad)

    p1 = _edge_agg_partials(feats, src3, dst3)
    h1 = _gin_mlp(feats, p1, W1a, b1a, g1, be1, W1b, b1b)

    p2 = _edge_agg_partials(h1, src3, dst3)
    h2 = _gin_mlp(h1, p2, W2a, b2a, g2, be2, W2b, b2b)

    out = _pool_head(h1, h2, batch, lin1_W, lin1_b, lin2_W, lin2_b)
    return (out, feats)


# speedup vs baseline: 11.8739x; 1.0098x over previous
import functools

import jax
import jax.numpy as jnp
from jax import lax
from jax.experimental import pallas as pl
from jax.experimental.pallas import tpu as pltpu
from jax.experimental.pallas import tpu_sc as plsc

N = 10000
E = 320000
D = 128
G = 64
OUT = 16
VPAD = 512

NC = 2
NS = 16
NW = NC * NS
EPW = E // NW
CH = 128
NCHUNK = 80
EPWP = NCHUNK * CH
PADE = EPWP - EPW
NB = 2
SPLIT = 4
IW = 16
NWIN = NCHUNK // IW
NP = 10240
RPT = NP // NS


def _edge_agg_partials(feats, src3, dst3):
    mesh = plsc.VectorSubcoreMesh(core_axis_name="c", subcore_axis_name="s")

    @functools.partial(
        pl.kernel,
        out_type=jax.ShapeDtypeStruct((NC, NP, D), jnp.float32),
        mesh=mesh,
        scratch_types=[
            pltpu.VMEM((2, IW, CH), jnp.int32),
            pltpu.VMEM((2, IW, CH), jnp.int32),
            pltpu.VMEM((SPLIT * NB, CH // SPLIT, D), jnp.float32),
            pltpu.VMEM_SHARED((NP, D), jnp.float32),
            pltpu.SemaphoreType.DMA,
            pltpu.SemaphoreType.DMA,
        ] + [pltpu.SemaphoreType.DMA] * (2 * SPLIT * NB),
    )
    def k(feats_hbm, src_hbm, dst_hbm, out_hbm, srcw, dstw, rows, acc,
          isem0, isem1, *sems):
        isem = (isem0, isem1)
        gsem = sems[:SPLIT * NB]
        ssem = sems[SPLIT * NB:]
        cid = lax.axis_index("c")
        sid = lax.axis_index("s")
        wid = cid * NS + sid

        def idx_fetch(w):
            p = w % 2
            pltpu.async_copy(src_hbm.at[wid, pl.ds(w * IW, IW)], srcw.at[p],
                             isem[p])
            pltpu.async_copy(dst_hbm.at[wid, pl.ds(w * IW, IW)], dstw.at[p],
                             isem[p])

        def idx_wait(w):
            p = w % 2
            pltpu.make_async_copy(src_hbm.at[wid, pl.ds(w * IW, IW)],
                                  srcw.at[p], isem[p]).wait()
            pltpu.make_async_copy(dst_hbm.at[wid, pl.ds(w * IW, IW)],
                                  dstw.at[p], isem[p]).wait()

        idx_fetch(0)

        zrow = rows.at[0]
        @pl.loop(0, CH // SPLIT)
        def _(i):
            @pl.loop(0, D, step=16)
            def _(j):
                zrow.at[i, pl.ds(j, 16)][...] = jnp.zeros((16,), jnp.float32)

        @pl.loop(0, RPT, step=CH // SPLIT)
        def _(r):
            pltpu.sync_copy(zrow, acc.at[pl.ds(sid * RPT + r, CH // SPLIT)])

        plsc.subcore_barrier()

        HC = CH // SPLIT
        NSLOT = SPLIT * NB

        def subidx(arr, p, sj):
            return arr.at[p, sj // SPLIT, pl.ds((sj % SPLIT) * HC, HC)]

        def gather(p, sj, b):
            pltpu.async_copy(feats_hbm.at[subidx(srcw, p, sj)], rows.at[b],
                             gsem[b])

        def gather_wait(p, sj, b):
            pltpu.make_async_copy(
                feats_hbm.at[subidx(srcw, p, sj)], rows.at[b], gsem[b]).wait()

        def scat(p, sj, b):
            pltpu.async_copy(rows.at[b], acc.at[subidx(dstw, p, sj)], ssem[b],
                             add=True)

        def scat_wait(p, sj, b):
            pltpu.make_async_copy(
                rows.at[b], acc.at[subidx(dstw, p, sj)], ssem[b]).wait()

        NSJ = SPLIT * IW
        for w in range(NWIN):
            p = w % 2
            idx_wait(w)
            if w + 1 < NWIN:
                idx_fetch(w + 1)
            for b in range(NSLOT):
                gather(p, b, b)

            @pl.loop(0, NSJ // NSLOT - 1)
            def _(g):
                base = g * NSLOT
                for b in range(NSLOT):
                    gather_wait(p, base + b, b)
                    scat(p, base + b, b)
                for b in range(NSLOT):
                    scat_wait(p, base + b, b)
                    gather(p, base + NSLOT + b, b)

            last = NSJ - NSLOT
            for b in range(NSLOT):
                gather_wait(p, last + b, b)
                scat(p, last + b, b)
            for b in range(NSLOT):
                scat_wait(p, last + b, b)

        plsc.subcore_barrier()

        @pl.loop(0, RPT, step=CH)
        def _(r):
            pltpu.sync_copy(
                acc.at[pl.ds(sid * RPT + r, CH)],
                out_hbm.at[cid, pl.ds(sid * RPT + r, CH)],
            )

    return k(feats, src3, dst3)


def _embed_kernel(x_ref, emb_ref, out_ref):
    xb = x_ref[0]
    ids = jax.lax.broadcasted_iota(jnp.int32, (VPAD, xb.shape[1]), 0)
    onehot = (ids == xb).astype(jnp.float32)
    out_ref[...] = jax.lax.dot_general(
        onehot, emb_ref[...], (((0,), (0,)), ((), ())),
        preferred_element_type=jnp.float32)


def _embed(x, emb_pad):
    blk = 1000
    x3 = x.reshape(N // blk, 1, blk)
    return pl.pallas_call(
        _embed_kernel,
        out_shape=jax.ShapeDtypeStruct((N, D), jnp.float32),
        grid=(N // blk,),
        in_specs=[
            pl.BlockSpec((1, 1, blk), lambda i: (i, 0, 0)),
            pl.BlockSpec((VPAD, D), lambda i: (0, 0)),
        ],
        out_specs=pl.BlockSpec((blk, D), lambda i: (i, 0)),
    )(x3, emb_pad)


def _mlp_kernel(x_ref, p_ref, Wa_ref, ba_ref, g_ref, be_ref, Wb_ref, bb_ref,
                out_ref):
    hin = x_ref[...] + p_ref[0, :N] + p_ref[1, :N]
    t = jnp.dot(hin, Wa_ref[...], preferred_element_type=jnp.float32)
    t = t + ba_ref[...]
    mu = jnp.mean(t, axis=0, keepdims=True)
    var = jnp.mean(jnp.square(t - mu), axis=0, keepdims=True)
    tn = g_ref[...] * (t - mu) * jax.lax.rsqrt(var + 1e-5) + be_ref[...]
    r = jnp.maximum(tn, 0.0)
    h = jnp.dot(r, Wb_ref[...], preferred_element_type=jnp.float32)
    out_ref[...] = jnp.maximum(h + bb_ref[...], 0.0)


def _gin_mlp(xin, partials, Wa, ba, g, be, Wb, bb):
    return pl.pallas_call(
        _mlp_kernel,
        out_shape=jax.ShapeDtypeStruct((N, D), jnp.float32),
    )(xin, partials, Wa, ba.reshape(1, D), g.reshape(1, D), be.reshape(1, D),
      Wb, bb.reshape(1, D))


def _pool_kernel(h1_ref, h2_ref, b_ref, W1_ref, b1_ref, W2_ref, b2_ref,
                 out_ref):
    gids = jax.lax.broadcasted_iota(jnp.int32, (G, N), 0)
    onehot = (gids == b_ref[...]).astype(jnp.float32)
    hg1 = jnp.dot(onehot, h1_ref[...], preferred_element_type=jnp.float32)
    hg2 = jnp.dot(onehot, h2_ref[...], preferred_element_type=jnp.float32)
    hg = jnp.concatenate([hg1, hg2], axis=1)
    y = jnp.dot(hg, W1_ref[...], preferred_element_type=jnp.float32)
    y = jnp.maximum(y + b1_ref[...], 0.0)
    out_ref[...] = jnp.dot(y, W2_ref[...],
                           preferred_element_type=jnp.float32) + b2_ref[...]


def _pool_head(h1, h2, batch, lin1_W, lin1_b, lin2_W, lin2_b):
    return pl.pallas_call(
        _pool_kernel,
        out_shape=jax.ShapeDtypeStruct((G, OUT), jnp.float32),
    )(h1, h2, batch.reshape(1, N), lin1_W, lin1_b.reshape(1, D), lin2_W,
      lin2_b.reshape(1, OUT))


def kernel(x, edge_index, batch, emb, W1a, b1a, g1, be1, W1b, b1b,
           W2a, b2a, g2, be2, W2b, b2b, lin1_W, lin1_b, lin2_W, lin2_b):
    emb_pad = jnp.zeros((VPAD, D), jnp.float32).at[:emb.shape[0]].set(emb)
    pad_src = jnp.broadcast_to((37 * jnp.arange(PADE, dtype=jnp.int32)) % N,
                               (NW, PADE))
    pad_dst = jnp.broadcast_to(N + jnp.arange(PADE, dtype=jnp.int32),
                               (NW, PADE))
    src3 = jnp.concatenate(
        [edge_index[0].reshape(NW, EPW), pad_src], axis=1
    ).reshape(NW, NCHUNK, CH)
    dst3 = jnp.concatenate(
        [edge_index[1].reshape(NW, EPW), pad_dst], axis=1
    ).reshape(NW, NCHUNK, CH)

    feats = _embed(x.astype(jnp.int32), emb_pad)

    p1 = _edge_agg_partials(feats, src3, dst3)
    h1 = _gin_mlp(feats, p1, W1a, b1a, g1, be1, W1b, b1b)

    p2 = _edge_agg_partials(h1, src3, dst3)
    h2 = _gin_mlp(h1, p2, W2a, b2a, g2, be2, W2b, b2b)

    out = _pool_head(h1, h2, batch, lin1_W, lin1_b, lin2_W, lin2_b)
    return (out, feats)


# speedup vs baseline: 12.1779x; 1.0256x over previous
import functools

import jax
import jax.numpy as jnp
from jax import lax
from jax.experimental import pallas as pl
from jax.experimental.pallas import tpu as pltpu
from jax.experimental.pallas import tpu_sc as plsc

N = 10000
E = 320000
D = 128
G = 64
OUT = 16
VPAD = 512

NC = 2
NS = 16
NW = NC * NS
EPW = E // NW
WE = 2048
NFW = EPW // WE
TAIL = EPW - NFW * WE
HC = 32
NSLOT = 8
TSJ = TAIL // HC
TREM = TAIL - TSJ * HC
NP = 10240
RPT = NP // NS


def _edge_agg_partials(feats, src3, dst3):
    mesh = plsc.VectorSubcoreMesh(core_axis_name="c", subcore_axis_name="s")

    @functools.partial(
        pl.kernel,
        out_type=jax.ShapeDtypeStruct((NC, NP, D), jnp.float32),
        mesh=mesh,
        scratch_types=[
            pltpu.VMEM((WE,), jnp.int32),
            pltpu.VMEM((WE,), jnp.int32),
            pltpu.VMEM((WE,), jnp.int32),
            pltpu.VMEM((WE,), jnp.int32),
            pltpu.VMEM((NSLOT, HC, D), jnp.float32),
            pltpu.VMEM_SHARED((NP, D), jnp.float32),
            pltpu.SemaphoreType.DMA,
            pltpu.SemaphoreType.DMA,
        ] + [pltpu.SemaphoreType.DMA] * (2 * NSLOT),
    )
    def k(feats_hbm, src_hbm, dst_hbm, out_hbm, srcw0, srcw1, dstw0, dstw1,
          rows, acc, isem0, isem1, *sems):
        idxb = ((srcw0, dstw0), (srcw1, dstw1))
        isem = (isem0, isem1)
        gsem = sems[:NSLOT]
        ssem = sems[NSLOT:]
        cid = lax.axis_index("c")
        sid = lax.axis_index("s")
        wid = cid * NS + sid

        def idx_copies(w):
            p = w % 2
            size = WE if w < NFW else TAIL
            sw, dw = idxb[p]
            return (
                pltpu.make_async_copy(
                    src_hbm.at[wid, 0, pl.ds(w * WE, size)],
                    sw.at[pl.ds(0, size)], isem[p]),
                pltpu.make_async_copy(
                    dst_hbm.at[wid, 0, pl.ds(w * WE, size)],
                    dw.at[pl.ds(0, size)], isem[p]),
            )

        def idx_fetch(w):
            for c in idx_copies(w):
                c.start()

        def idx_wait(w):
            for c in idx_copies(w):
                c.wait()

        idx_fetch(0)

        zrow = rows.at[0]
        @pl.loop(0, HC)
        def _(i):
            @pl.loop(0, D, step=16)
            def _(j):
                zrow.at[i, pl.ds(j, 16)][...] = jnp.zeros((16,), jnp.float32)

        @pl.loop(0, RPT, step=HC)
        def _(r):
            pltpu.sync_copy(zrow, acc.at[pl.ds(sid * RPT + r, HC)])

        plsc.subcore_barrier()

        def gather(p, sj, b):
            pltpu.async_copy(feats_hbm.at[idxb[p][0].at[pl.ds(sj * HC, HC)]],
                             rows.at[b], gsem[b])

        def gather_wait(p, sj, b):
            pltpu.make_async_copy(
                feats_hbm.at[idxb[p][0].at[pl.ds(sj * HC, HC)]],
                rows.at[b], gsem[b]).wait()

        def scat(p, sj, b):
            pltpu.async_copy(rows.at[b],
                             acc.at[idxb[p][1].at[pl.ds(sj * HC, HC)]],
                             ssem[b], add=True)

        def scat_wait(p, sj, b):
            pltpu.make_async_copy(
                rows.at[b], acc.at[idxb[p][1].at[pl.ds(sj * HC, HC)]],
                ssem[b]).wait()

        for w in range(NFW + 1):
            p = w % 2
            idx_wait(w)
            if w < NFW:
                idx_fetch(w + 1)
            nsj = (WE if w < NFW else TSJ * HC) // HC
            for b in range(NSLOT):
                gather(p, b, b)

            @pl.loop(0, nsj // NSLOT - 1)
            def _(g):
                base = g * NSLOT
                for b in range(NSLOT):
                    gather_wait(p, base + b, b)
                    scat(p, base + b, b)
                for b in range(NSLOT):
                    scat_wait(p, base + b, b)
                    gather(p, base + NSLOT + b, b)

            last = nsj - NSLOT
            for b in range(NSLOT):
                gather_wait(p, last + b, b)
                scat(p, last + b, b)
            for b in range(NSLOT):
                scat_wait(p, last + b, b)

        pt = NFW % 2
        sw, dw = idxb[pt]
        pltpu.sync_copy(feats_hbm.at[sw.at[pl.ds(TSJ * HC, TREM)]],
                        rows.at[0, pl.ds(0, TREM)])
        pltpu.sync_copy(rows.at[0, pl.ds(0, TREM)],
                        acc.at[dw.at[pl.ds(TSJ * HC, TREM)]], add=True)

        plsc.subcore_barrier()

        @pl.loop(0, RPT, step=128)
        def _(r):
            pltpu.sync_copy(
                acc.at[pl.ds(sid * RPT + r, 128)],
                out_hbm.at[cid, pl.ds(sid * RPT + r, 128)],
            )

    return k(feats, src3, dst3)


def _embed_kernel(x_ref, emb_ref, out_ref):
    xb = x_ref[0]
    ids = jax.lax.broadcasted_iota(jnp.int32, (VPAD, xb.shape[1]), 0)
    onehot = (ids == xb).astype(jnp.float32)
    out_ref[...] = jax.lax.dot_general(
        onehot, emb_ref[...], (((0,), (0,)), ((), ())),
        preferred_element_type=jnp.float32)


def _embed(x, emb_pad):
    blk = 1000
    x3 = x.reshape(N // blk, 1, blk)
    return pl.pallas_call(
        _embed_kernel,
        out_shape=jax.ShapeDtypeStruct((N, D), jnp.float32),
        grid=(N // blk,),
        in_specs=[
            pl.BlockSpec((1, 1, blk), lambda i: (i, 0, 0)),
            pl.BlockSpec((VPAD, D), lambda i: (0, 0)),
        ],
        out_specs=pl.BlockSpec((blk, D), lambda i: (i, 0)),
    )(x3, emb_pad)


def _gin_mlp_body(hin, Wa_ref, ba_ref, g_ref, be_ref, Wb_ref, bb_ref):
    t = jnp.dot(hin, Wa_ref[...], preferred_element_type=jnp.float32)
    t = t + ba_ref[...]
    mu = jnp.mean(t, axis=0, keepdims=True)
    var = jnp.mean(jnp.square(t - mu), axis=0, keepdims=True)
    tn = g_ref[...] * (t - mu) * jax.lax.rsqrt(var + 1e-5) + be_ref[...]
    r = jnp.maximum(tn, 0.0)
    h = jnp.dot(r, Wb_ref[...], preferred_element_type=jnp.float32)
    return jnp.maximum(h + bb_ref[...], 0.0)


def _mlp1_kernel(x_ref, p_ref, Wa_ref, ba_ref, g_ref, be_ref, Wb_ref, bb_ref,
                 out_ref):
    hin = x_ref[...] + p_ref[0, :N] + p_ref[1, :N]
    out_ref[...] = _gin_mlp_body(hin, Wa_ref, ba_ref, g_ref, be_ref, Wb_ref,
                                 bb_ref)


def _gin_mlp1(xin, partials, Wa, ba, g, be, Wb, bb):
    return pl.pallas_call(
        _mlp1_kernel,
        out_shape=jax.ShapeDtypeStruct((N, D), jnp.float32),
    )(xin, partials, Wa, ba.reshape(1, D), g.reshape(1, D), be.reshape(1, D),
      Wb, bb.reshape(1, D))


def _mlp2_pool_kernel(h1_ref, p_ref, Wa_ref, ba_ref, g_ref, be_ref, Wb_ref,
                      bb_ref, b_ref, W1_ref, b1_ref, W2_ref, b2_ref, out_ref):
    hin = h1_ref[...] + p_ref[0, :N] + p_ref[1, :N]
    h2 = _gin_mlp_body(hin, Wa_ref, ba_ref, g_ref, be_ref, Wb_ref, bb_ref)
    gids = jax.lax.broadcasted_iota(jnp.int32, (G, N), 0)
    onehot = (gids == b_ref[...]).astype(jnp.float32)
    hg1 = jnp.dot(onehot, h1_ref[...], preferred_element_type=jnp.float32)
    hg2 = jnp.dot(onehot, h2, preferred_element_type=jnp.float32)
    hg = jnp.concatenate([hg1, hg2], axis=1)
    y = jnp.dot(hg, W1_ref[...], preferred_element_type=jnp.float32)
    y = jnp.maximum(y + b1_ref[...], 0.0)
    out_ref[...] = jnp.dot(y, W2_ref[...],
                           preferred_element_type=jnp.float32) + b2_ref[...]


def _mlp2_pool(h1, partials, Wa, ba, g, be, Wb, bb, batch,
               lin1_W, lin1_b, lin2_W, lin2_b):
    return pl.pallas_call(
        _mlp2_pool_kernel,
        out_shape=jax.ShapeDtypeStruct((G, OUT), jnp.float32),
    )(h1, partials, Wa, ba.reshape(1, D), g.reshape(1, D), be.reshape(1, D),
      Wb, bb.reshape(1, D), batch.reshape(1, N), lin1_W,
      lin1_b.reshape(1, D), lin2_W, lin2_b.reshape(1, OUT))


def kernel(x, edge_index, batch, emb, W1a, b1a, g1, be1, W1b, b1b,
           W2a, b2a, g2, be2, W2b, b2b, lin1_W, lin1_b, lin2_W, lin2_b):
    emb_pad = jnp.zeros((VPAD, D), jnp.float32).at[:emb.shape[0]].set(emb)
    src3 = edge_index[0].reshape(NW, 1, EPW)
    dst3 = edge_index[1].reshape(NW, 1, EPW)

    feats = _embed(x.astype(jnp.int32), emb_pad)

    p1 = _edge_agg_partials(feats, src3, dst3)
    h1 = _gin_mlp1(feats, p1, W1a, b1a, g1, be1, W1b, b1b)

    p2 = _edge_agg_partials(h1, src3, dst3)
    out = _mlp2_pool(h1, p2, W2a, b2a, g2, be2, W2b, b2b, batch,
                     lin1_W, lin1_b, lin2_W, lin2_b)
    return (out, feats)


# speedup vs baseline: 12.7195x; 1.0445x over previous
import functools

import jax
import jax.numpy as jnp
from jax import lax
from jax.experimental import pallas as pl
from jax.experimental.pallas import tpu as pltpu
from jax.experimental.pallas import tpu_sc as plsc

N = 10000
E = 320000
D = 128
G = 64
OUT = 16
VPAD = 512

NC = 2
NS = 16
NW = NC * NS
EPW = E // NW
WE = 2048
NFW = EPW // WE
TAIL = EPW - NFW * WE
FS = WE + 128
FST = 1920
HC = 32
NSLOT = 8
TSJ = TAIL // HC
TREM = TAIL - TSJ * HC
NP = 10240
RPT = NP // NS


def _edge_agg_partials(feats, eidx):
    mesh = plsc.VectorSubcoreMesh(core_axis_name="c", subcore_axis_name="s")

    @functools.partial(
        pl.kernel,
        out_type=jax.ShapeDtypeStruct((NC, NP, D), jnp.float32),
        mesh=mesh,
        scratch_types=[
            pltpu.VMEM((FS,), jnp.int32),
            pltpu.VMEM((FS,), jnp.int32),
            pltpu.VMEM((FS,), jnp.int32),
            pltpu.VMEM((FS,), jnp.int32),
            pltpu.VMEM((NSLOT, HC, D), jnp.float32),
            pltpu.VMEM_SHARED((NP, D), jnp.float32),
            pltpu.SemaphoreType.DMA,
            pltpu.SemaphoreType.DMA,
        ] + [pltpu.SemaphoreType.DMA] * (2 * NSLOT),
    )
    def k(feats_hbm, eidx_hbm, out_hbm, srcw0, srcw1, dstw0, dstw1,
          rows, acc, isem0, isem1, *sems):
        idxb = ((srcw0, dstw0), (srcw1, dstw1))
        isem = (isem0, isem1)
        gsem = sems[:NSLOT]
        ssem = sems[NSLOT:]
        cid = lax.axis_index("c")
        sid = lax.axis_index("s")
        wid = cid * NS + sid
        off = (wid % 8) * 16

        def idx_copies(w):
            p = w % 2
            size = FS if w < NFW else FST
            astart = pl.multiple_of(wid * EPW - off + w * WE, 128)
            sw, dw = idxb[p]
            return (
                pltpu.make_async_copy(
                    eidx_hbm.at[0, 0, pl.ds(astart, size)],
                    sw.at[pl.ds(0, size)], isem[p]),
                pltpu.make_async_copy(
                    eidx_hbm.at[1, 0, pl.ds(astart, size)],
                    dw.at[pl.ds(0, size)], isem[p]),
            )

        def idx_fetch(w):
            for c in idx_copies(w):
                c.start()

        def idx_wait(w):
            for c in idx_copies(w):
                c.wait()

        idx_fetch(0)

        zrow = rows.at[0]
        @pl.loop(0, HC)
        def _(i):
            @pl.loop(0, D, step=16)
            def _(j):
                zrow.at[i, pl.ds(j, 16)][...] = jnp.zeros((16,), jnp.float32)

        @pl.loop(0, RPT, step=HC)
        def _(r):
            pltpu.sync_copy(zrow, acc.at[pl.ds(sid * RPT + r, HC)])

        plsc.subcore_barrier()

        def gather(p, sj, b):
            pltpu.async_copy(
                feats_hbm.at[idxb[p][0].at[pl.ds(off + sj * HC, HC)]],
                rows.at[b], gsem[b])

        def gather_wait(p, sj, b):
            pltpu.make_async_copy(
                feats_hbm.at[idxb[p][0].at[pl.ds(off + sj * HC, HC)]],
                rows.at[b], gsem[b]).wait()

        def scat(p, sj, b):
            pltpu.async_copy(rows.at[b],
                             acc.at[idxb[p][1].at[pl.ds(off + sj * HC, HC)]],
                             ssem[b], add=True)

        def scat_wait(p, sj, b):
            pltpu.make_async_copy(
                rows.at[b], acc.at[idxb[p][1].at[pl.ds(off + sj * HC, HC)]],
                ssem[b]).wait()

        for w in range(NFW + 1):
            p = w % 2
            idx_wait(w)
            if w < NFW:
                idx_fetch(w + 1)
            nsj = (WE if w < NFW else TSJ * HC) // HC
            for b in range(NSLOT):
                gather(p, b, b)

            @pl.loop(0, nsj // NSLOT - 1)
            def _(g):
                base = g * NSLOT
                for b in range(NSLOT):
                    gather_wait(p, base + b, b)
                    scat(p, base + b, b)
                for b in range(NSLOT):
                    scat_wait(p, base + b, b)
                    gather(p, base + NSLOT + b, b)

            last = nsj - NSLOT
            for b in range(NSLOT):
                gather_wait(p, last + b, b)
                scat(p, last + b, b)
            for b in range(NSLOT):
                scat_wait(p, last + b, b)

        pt = NFW % 2
        sw, dw = idxb[pt]
        pltpu.sync_copy(feats_hbm.at[sw.at[pl.ds(off + TSJ * HC, TREM)]],
                        rows.at[0, pl.ds(0, TREM)])
        pltpu.sync_copy(rows.at[0, pl.ds(0, TREM)],
                        acc.at[dw.at[pl.ds(off + TSJ * HC, TREM)]], add=True)

        plsc.subcore_barrier()

        @pl.loop(0, RPT, step=128)
        def _(r):
            pltpu.sync_copy(
                acc.at[pl.ds(sid * RPT + r, 128)],
                out_hbm.at[cid, pl.ds(sid * RPT + r, 128)],
            )

    return k(feats, eidx)


def _embed_kernel(x_ref, emb_ref, out_ref):
    xb = x_ref[0]
    ids = jax.lax.broadcasted_iota(jnp.int32, (VPAD, xb.shape[1]), 0)
    onehot = (ids == xb).astype(jnp.float32)
    out_ref[...] = jax.lax.dot_general(
        onehot, emb_ref[...], (((0,), (0,)), ((), ())),
        preferred_element_type=jnp.float32)


def _embed(x, emb_pad):
    blk = 1000
    x3 = x.reshape(N // blk, 1, blk)
    return pl.pallas_call(
        _embed_kernel,
        out_shape=jax.ShapeDtypeStruct((N, D), jnp.float32),
        grid=(N // blk,),
        in_specs=[
            pl.BlockSpec((1, 1, blk), lambda i: (i, 0, 0)),
            pl.BlockSpec((VPAD, D), lambda i: (0, 0)),
        ],
        out_specs=pl.BlockSpec((blk, D), lambda i: (i, 0)),
    )(x3, emb_pad)


def _gin_mlp_body(hin, Wa_ref, ba_ref, g_ref, be_ref, Wb_ref, bb_ref):
    t = jnp.dot(hin, Wa_ref[...], preferred_element_type=jnp.float32)
    t = t + ba_ref[...]
    mu = jnp.mean(t, axis=0, keepdims=True)
    var = jnp.mean(jnp.square(t - mu), axis=0, keepdims=True)
    tn = g_ref[...] * (t - mu) * jax.lax.rsqrt(var + 1e-5) + be_ref[...]
    r = jnp.maximum(tn, 0.0)
    h = jnp.dot(r, Wb_ref[...], preferred_element_type=jnp.float32)
    return jnp.maximum(h + bb_ref[...], 0.0)


def _mlp1_kernel(x_ref, p_ref, Wa_ref, ba_ref, g_ref, be_ref, Wb_ref, bb_ref,
                 out_ref):
    hin = x_ref[...] + p_ref[0, :N] + p_ref[1, :N]
    out_ref[...] = _gin_mlp_body(hin, Wa_ref, ba_ref, g_ref, be_ref, Wb_ref,
                                 bb_ref)


def _gin_mlp1(xin, partials, Wa, ba, g, be, Wb, bb):
    return pl.pallas_call(
        _mlp1_kernel,
        out_shape=jax.ShapeDtypeStruct((N, D), jnp.float32),
    )(xin, partials, Wa, ba.reshape(1, D), g.reshape(1, D), be.reshape(1, D),
      Wb, bb.reshape(1, D))


def _mlp2_pool_kernel(h1_ref, p_ref, Wa_ref, ba_ref, g_ref, be_ref, Wb_ref,
                      bb_ref, b_ref, W1_ref, b1_ref, W2_ref, b2_ref, out_ref):
    hin = h1_ref[...] + p_ref[0, :N] + p_ref[1, :N]
    h2 = _gin_mlp_body(hin, Wa_ref, ba_ref, g_ref, be_ref, Wb_ref, bb_ref)
    gids = jax.lax.broadcasted_iota(jnp.int32, (G, N), 0)
    onehot = (gids == b_ref[...]).astype(jnp.float32)
    hg1 = jnp.dot(onehot, h1_ref[...], preferred_element_type=jnp.float32)
    hg2 = jnp.dot(onehot, h2, preferred_element_type=jnp.float32)
    hg = jnp.concatenate([hg1, hg2], axis=1)
    y = jnp.dot(hg, W1_ref[...], preferred_element_type=jnp.float32)
    y = jnp.maximum(y + b1_ref[...], 0.0)
    out_ref[...] = jnp.dot(y, W2_ref[...],
                           preferred_element_type=jnp.float32) + b2_ref[...]


def _mlp2_pool(h1, partials, Wa, ba, g, be, Wb, bb, batch,
               lin1_W, lin1_b, lin2_W, lin2_b):
    return pl.pallas_call(
        _mlp2_pool_kernel,
        out_shape=jax.ShapeDtypeStruct((G, OUT), jnp.float32),
    )(h1, partials, Wa, ba.reshape(1, D), g.reshape(1, D), be.reshape(1, D),
      Wb, bb.reshape(1, D), batch.reshape(1, N), lin1_W,
      lin1_b.reshape(1, D), lin2_W, lin2_b.reshape(1, OUT))


def kernel(x, edge_index, batch, emb, W1a, b1a, g1, be1, W1b, b1b,
           W2a, b2a, g2, be2, W2b, b2b, lin1_W, lin1_b, lin2_W, lin2_b):
    emb_pad = jnp.zeros((VPAD, D), jnp.float32).at[:emb.shape[0]].set(emb)
    eidx = edge_index.reshape(2, 1, E)

    feats = _embed(x.astype(jnp.int32), emb_pad)

    p1 = _edge_agg_partials(feats, eidx)
    h1 = _gin_mlp1(feats, p1, W1a, b1a, g1, be1, W1b, b1b)

    p2 = _edge_agg_partials(h1, eidx)
    out = _mlp2_pool(h1, p2, W2a, b2a, g2, be2, W2b, b2b, batch,
                     lin1_W, lin1_b, lin2_W, lin2_b)
    return (out, feats)


# speedup vs baseline: 12.7250x; 1.0004x over previous
import functools

import jax
import jax.numpy as jnp
from jax import lax
from jax.experimental import pallas as pl
from jax.experimental.pallas import tpu as pltpu
from jax.experimental.pallas import tpu_sc as plsc

N = 10000
E = 320000
D = 128
G = 64
OUT = 16
VPAD = 512

NC = 2
NS = 16
NW = NC * NS
EPW = E // NW
WE = 2048
NFW = EPW // WE
TAIL = EPW - NFW * WE
FS = WE + 128
FST = 1920
HC = 32
NSLOT = 8
TSJ = TAIL // HC
TREM = TAIL - TSJ * HC
NP = 10240
RPT = NP // NS


def _edge_agg_partials(feats, eidx):
    mesh = plsc.VectorSubcoreMesh(core_axis_name="c", subcore_axis_name="s")

    @functools.partial(
        pl.kernel,
        out_type=jax.ShapeDtypeStruct((NC, NP, D), jnp.float32),
        mesh=mesh,
        scratch_types=[
            pltpu.VMEM((FS,), jnp.int32),
            pltpu.VMEM((FS,), jnp.int32),
            pltpu.VMEM((FS,), jnp.int32),
            pltpu.VMEM((FS,), jnp.int32),
            pltpu.VMEM((NSLOT, HC, D), jnp.float32),
            pltpu.VMEM_SHARED((NP, D), jnp.float32),
            pltpu.SemaphoreType.DMA,
            pltpu.SemaphoreType.DMA,
        ] + [pltpu.SemaphoreType.DMA] * (2 * NSLOT),
    )
    def k(feats_hbm, eidx_hbm, out_hbm, srcw0, srcw1, dstw0, dstw1,
          rows, acc, isem0, isem1, *sems):
        idxb = ((srcw0, dstw0), (srcw1, dstw1))
        isem = (isem0, isem1)
        gsem = sems[:NSLOT]
        ssem = sems[NSLOT:]
        cid = lax.axis_index("c")
        sid = lax.axis_index("s")
        wid = cid * NS + sid
        off = (wid % 8) * 16

        def idx_copies(w):
            p = w % 2
            size = FS if w < NFW else FST
            astart = pl.multiple_of(wid * EPW - off + w * WE, 128)
            sw, dw = idxb[p]
            return (
                pltpu.make_async_copy(
                    eidx_hbm.at[0, 0, pl.ds(astart, size)],
                    sw.at[pl.ds(0, size)], isem[p]),
                pltpu.make_async_copy(
                    eidx_hbm.at[1, 0, pl.ds(astart, size)],
                    dw.at[pl.ds(0, size)], isem[p]),
            )

        def idx_fetch(w):
            for c in idx_copies(w):
                c.start()

        def idx_wait(w):
            for c in idx_copies(w):
                c.wait()

        idx_fetch(0)

        zrow = rows.at[0]
        @pl.loop(0, HC)
        def _(i):
            @pl.loop(0, D, step=16)
            def _(j):
                zrow.at[i, pl.ds(j, 16)][...] = jnp.zeros((16,), jnp.float32)

        @pl.loop(0, RPT, step=HC)
        def _(r):
            pltpu.sync_copy(zrow, acc.at[pl.ds(sid * RPT + r, HC)])

        plsc.subcore_barrier()

        def gather(p, sj, b):
            pltpu.async_copy(
                feats_hbm.at[idxb[p][0].at[pl.ds(off + sj * HC, HC)]],
                rows.at[b], gsem[b])

        def gather_wait(p, sj, b):
            pltpu.make_async_copy(
                feats_hbm.at[idxb[p][0].at[pl.ds(off + sj * HC, HC)]],
                rows.at[b], gsem[b]).wait()

        def scat(p, sj, b):
            pltpu.async_copy(rows.at[b],
                             acc.at[idxb[p][1].at[pl.ds(off + sj * HC, HC)]],
                             ssem[b], add=True)

        def scat_wait(p, sj, b):
            pltpu.make_async_copy(
                rows.at[b], acc.at[idxb[p][1].at[pl.ds(off + sj * HC, HC)]],
                ssem[b]).wait()

        for w in range(NFW + 1):
            p = w % 2
            idx_wait(w)
            if w < NFW:
                idx_fetch(w + 1)
            nsj = (WE if w < NFW else TSJ * HC) // HC
            for b in range(NSLOT):
                gather(p, b, b)

            @pl.loop(0, nsj // NSLOT - 1)
            def _(g):
                base = g * NSLOT
                for b in range(NSLOT):
                    gather_wait(p, base + b, b)
                    scat(p, base + b, b)
                for b in range(NSLOT):
                    scat_wait(p, base + b, b)
                    gather(p, base + NSLOT + b, b)

            last = nsj - NSLOT
            for b in range(NSLOT):
                gather_wait(p, last + b, b)
                scat(p, last + b, b)
            for b in range(NSLOT):
                scat_wait(p, last + b, b)

        pt = NFW % 2
        sw, dw = idxb[pt]
        pltpu.sync_copy(feats_hbm.at[sw.at[pl.ds(off + TSJ * HC, TREM)]],
                        rows.at[0, pl.ds(0, TREM)])
        pltpu.sync_copy(rows.at[0, pl.ds(0, TREM)],
                        acc.at[dw.at[pl.ds(off + TSJ * HC, TREM)]], add=True)

        plsc.subcore_barrier()

        @pl.loop(0, RPT, step=128)
        def _(r):
            pltpu.sync_copy(
                acc.at[pl.ds(sid * RPT + r, 128)],
                out_hbm.at[cid, pl.ds(sid * RPT + r, 128)],
            )

    return k(feats, eidx)


def _embed_kernel(x_ref, emb_ref, out_ref):
    xb = x_ref[0]
    ids = jax.lax.broadcasted_iota(jnp.int32, (VPAD, xb.shape[1]), 0)
    onehot = (ids == xb).astype(jnp.bfloat16)
    out_ref[...] = jax.lax.dot_general(
        onehot, emb_ref[...].astype(jnp.bfloat16), (((0,), (0,)), ((), ())),
        preferred_element_type=jnp.float32)


def _embed(x, emb_pad):
    blk = 1000
    x3 = x.reshape(N // blk, 1, blk)
    return pl.pallas_call(
        _embed_kernel,
        out_shape=jax.ShapeDtypeStruct((N, D), jnp.float32),
        grid=(N // blk,),
        in_specs=[
            pl.BlockSpec((1, 1, blk), lambda i: (i, 0, 0)),
            pl.BlockSpec((VPAD, D), lambda i: (0, 0)),
        ],
        out_specs=pl.BlockSpec((blk, D), lambda i: (i, 0)),
    )(x3, emb_pad)


def _gin_mlp_body(hin, Wa_ref, ba_ref, g_ref, be_ref, Wb_ref, bb_ref):
    t = jnp.dot(hin, Wa_ref[...], preferred_element_type=jnp.float32)
    t = t + ba_ref[...]
    mu = jnp.mean(t, axis=0, keepdims=True)
    var = jnp.mean(jnp.square(t - mu), axis=0, keepdims=True)
    tn = g_ref[...] * (t - mu) * jax.lax.rsqrt(var + 1e-5) + be_ref[...]
    r = jnp.maximum(tn, 0.0)
    h = jnp.dot(r, Wb_ref[...], preferred_element_type=jnp.float32)
    return jnp.maximum(h + bb_ref[...], 0.0)


def _mlp1_kernel(x_ref, p_ref, Wa_ref, ba_ref, g_ref, be_ref, Wb_ref, bb_ref,
                 out_ref):
    hin = x_ref[...] + p_ref[0, :N] + p_ref[1, :N]
    out_ref[...] = _gin_mlp_body(hin, Wa_ref, ba_ref, g_ref, be_ref, Wb_ref,
                                 bb_ref)


def _gin_mlp1(xin, partials, Wa, ba, g, be, Wb, bb):
    return pl.pallas_call(
        _mlp1_kernel,
        out_shape=jax.ShapeDtypeStruct((N, D), jnp.float32),
    )(xin, partials, Wa, ba.reshape(1, D), g.reshape(1, D), be.reshape(1, D),
      Wb, bb.reshape(1, D))


def _mlp2_pool_kernel(h1_ref, p_ref, Wa_ref, ba_ref, g_ref, be_ref, Wb_ref,
                      bb_ref, b_ref, W1_ref, b1_ref, W2_ref, b2_ref, out_ref):
    hin = h1_ref[...] + p_ref[0, :N] + p_ref[1, :N]
    h2 = _gin_mlp_body(hin, Wa_ref, ba_ref, g_ref, be_ref, Wb_ref, bb_ref)
    gids = jax.lax.broadcasted_iota(jnp.int32, (G, N), 0)
    onehot = (gids == b_ref[...]).astype(jnp.bfloat16)
    hg1 = jnp.dot(onehot, h1_ref[...].astype(jnp.bfloat16),
                  preferred_element_type=jnp.float32)
    hg2 = jnp.dot(onehot, h2.astype(jnp.bfloat16),
                  preferred_element_type=jnp.float32)
    hg = jnp.concatenate([hg1, hg2], axis=1)
    y = jnp.dot(hg, W1_ref[...], preferred_element_type=jnp.float32)
    y = jnp.maximum(y + b1_ref[...], 0.0)
    out_ref[...] = jnp.dot(y, W2_ref[...],
                           preferred_element_type=jnp.float32) + b2_ref[...]


def _mlp2_pool(h1, partials, Wa, ba, g, be, Wb, bb, batch,
               lin1_W, lin1_b, lin2_W, lin2_b):
    return pl.pallas_call(
        _mlp2_pool_kernel,
        out_shape=jax.ShapeDtypeStruct((G, OUT), jnp.float32),
    )(h1, partials, Wa, ba.reshape(1, D), g.reshape(1, D), be.reshape(1, D),
      Wb, bb.reshape(1, D), batch.reshape(1, N), lin1_W,
      lin1_b.reshape(1, D), lin2_W, lin2_b.reshape(1, OUT))


def kernel(x, edge_index, batch, emb, W1a, b1a, g1, be1, W1b, b1b,
           W2a, b2a, g2, be2, W2b, b2b, lin1_W, lin1_b, lin2_W, lin2_b):
    emb_pad = jnp.zeros((VPAD, D), jnp.float32).at[:emb.shape[0]].set(emb)
    eidx = edge_index.reshape(2, 1, E)

    feats = _embed(x.astype(jnp.int32), emb_pad)

    p1 = _edge_agg_partials(feats, eidx)
    h1 = _gin_mlp1(feats, p1, W1a, b1a, g1, be1, W1b, b1b)

    p2 = _edge_agg_partials(h1, eidx)
    out = _mlp2_pool(h1, p2, W2a, b2a, g2, be2, W2b, b2b, batch,
                     lin1_W, lin1_b, lin2_W, lin2_b)
    return (out, feats)


# speedup vs baseline: 12.7972x; 1.0057x over previous
import functools

import jax
import jax.numpy as jnp
from jax import lax
from jax.experimental import pallas as pl
from jax.experimental.pallas import tpu as pltpu
from jax.experimental.pallas import tpu_sc as plsc

N = 10000
E = 320000
D = 128
G = 64
OUT = 16
VPAD = 512

NC = 2
NS = 16
NW = NC * NS
EPW = E // NW
WE = 2048
NFW = EPW // WE
TAIL = EPW - NFW * WE
FS = WE + 128
FST = 1920
HC = 32
NSLOT = 8
TSJ = TAIL // HC
TREM = TAIL - TSJ * HC
NP = 10240
RPT = NP // NS


def _edge_agg_partials(feats, eidx):
    mesh = plsc.VectorSubcoreMesh(core_axis_name="c", subcore_axis_name="s")

    @functools.partial(
        pl.kernel,
        out_type=jax.ShapeDtypeStruct((NC, NP, D), jnp.float32),
        mesh=mesh,
        scratch_types=[
            pltpu.VMEM((FS,), jnp.int32),
            pltpu.VMEM((FS,), jnp.int32),
            pltpu.VMEM((FS,), jnp.int32),
            pltpu.VMEM((FS,), jnp.int32),
            pltpu.VMEM((NSLOT, HC, D), jnp.float32),
            pltpu.VMEM_SHARED((NP, D), jnp.float32),
            pltpu.SemaphoreType.DMA,
            pltpu.SemaphoreType.DMA,
        ] + [pltpu.SemaphoreType.DMA] * (2 * NSLOT),
    )
    def k(feats_hbm, eidx_hbm, out_hbm, srcw0, srcw1, dstw0, dstw1,
          rows, acc, isem0, isem1, *sems):
        idxb = ((srcw0, dstw0), (srcw1, dstw1))
        isem = (isem0, isem1)
        gsem = sems[:NSLOT]
        ssem = sems[NSLOT:]
        cid = lax.axis_index("c")
        sid = lax.axis_index("s")
        wid = cid * NS + sid
        off = (wid % 8) * 16

        def idx_copies(w):
            p = w % 2
            size = FS if w < NFW else FST
            astart = pl.multiple_of(wid * EPW - off + w * WE, 128)
            sw, dw = idxb[p]
            return (
                pltpu.make_async_copy(
                    eidx_hbm.at[pl.ds(astart, size)],
                    sw.at[pl.ds(0, size)], isem[p]),
                pltpu.make_async_copy(
                    eidx_hbm.at[pl.ds(pl.multiple_of(E + astart, 128), size)],
                    dw.at[pl.ds(0, size)], isem[p]),
            )

        def idx_fetch(w):
            for c in idx_copies(w):
                c.start()

        def idx_wait(w):
            for c in idx_copies(w):
                c.wait()

        idx_fetch(0)

        zrow = rows.at[0]
        @pl.loop(0, HC)
        def _(i):
            @pl.loop(0, D, step=16)
            def _(j):
                zrow.at[i, pl.ds(j, 16)][...] = jnp.zeros((16,), jnp.float32)

        @pl.loop(0, RPT, step=HC)
        def _(r):
            pltpu.async_copy(zrow, acc.at[pl.ds(sid * RPT + r, HC)], ssem[0])

        @pl.loop(0, RPT, step=HC)
        def _(r):
            pltpu.make_async_copy(
                zrow, acc.at[pl.ds(sid * RPT + r, HC)], ssem[0]).wait()

        plsc.subcore_barrier()

        def gather(p, sj, b):
            pltpu.async_copy(
                feats_hbm.at[idxb[p][0].at[pl.ds(off + sj * HC, HC)]],
                rows.at[b], gsem[b])

        def gather_wait(p, sj, b):
            pltpu.make_async_copy(
                feats_hbm.at[idxb[p][0].at[pl.ds(off + sj * HC, HC)]],
                rows.at[b], gsem[b]).wait()

        def scat(p, sj, b):
            pltpu.async_copy(rows.at[b],
                             acc.at[idxb[p][1].at[pl.ds(off + sj * HC, HC)]],
                             ssem[b], add=True)

        def scat_wait(p, sj, b):
            pltpu.make_async_copy(
                rows.at[b], acc.at[idxb[p][1].at[pl.ds(off + sj * HC, HC)]],
                ssem[b]).wait()

        for w in range(NFW + 1):
            p = w % 2
            idx_wait(w)
            if w < NFW:
                idx_fetch(w + 1)
            nsj = (WE if w < NFW else TSJ * HC) // HC
            for b in range(NSLOT):
                gather(p, b, b)

            @pl.loop(0, nsj // NSLOT - 1)
            def _(g):
                base = g * NSLOT
                for b in range(NSLOT):
                    gather_wait(p, base + b, b)
                    scat(p, base + b, b)
                for b in range(NSLOT):
                    scat_wait(p, base + b, b)
                    gather(p, base + NSLOT + b, b)

            last = nsj - NSLOT
            for b in range(NSLOT):
                gather_wait(p, last + b, b)
                scat(p, last + b, b)
            for b in range(NSLOT):
                scat_wait(p, last + b, b)

        pt = NFW % 2
        sw, dw = idxb[pt]
        pltpu.sync_copy(feats_hbm.at[sw.at[pl.ds(off + TSJ * HC, TREM)]],
                        rows.at[0, pl.ds(0, TREM)])
        pltpu.sync_copy(rows.at[0, pl.ds(0, TREM)],
                        acc.at[dw.at[pl.ds(off + TSJ * HC, TREM)]], add=True)

        plsc.subcore_barrier()

        @pl.loop(0, RPT, step=128)
        def _(r):
            pltpu.async_copy(
                acc.at[pl.ds(sid * RPT + r, 128)],
                out_hbm.at[cid, pl.ds(sid * RPT + r, 128)],
                ssem[0],
            )

        @pl.loop(0, RPT, step=128)
        def _(r):
            pltpu.make_async_copy(
                acc.at[pl.ds(sid * RPT + r, 128)],
                out_hbm.at[cid, pl.ds(sid * RPT + r, 128)],
                ssem[0],
            ).wait()

    return k(feats, eidx)


def _embed_kernel(x_ref, emb_ref, out_ref):
    xb = x_ref[0]
    ids = jax.lax.broadcasted_iota(jnp.int32, (VPAD, xb.shape[1]), 0)
    onehot = (ids == xb).astype(jnp.float32)
    out_ref[...] = jax.lax.dot_general(
        onehot, emb_ref[...], (((0,), (0,)), ((), ())),
        preferred_element_type=jnp.float32)


def _embed(x, emb_pad):
    blk = 1000
    x3 = x.reshape(N // blk, 1, blk)
    return pl.pallas_call(
        _embed_kernel,
        out_shape=jax.ShapeDtypeStruct((N, D), jnp.float32),
        grid=(N // blk,),
        in_specs=[
            pl.BlockSpec((1, 1, blk), lambda i: (i, 0, 0)),
            pl.BlockSpec((VPAD, D), lambda i: (0, 0)),
        ],
        out_specs=pl.BlockSpec((blk, D), lambda i: (i, 0)),
    )(x3, emb_pad)


def _gin_mlp_body(hin, Wa_ref, ba_ref, g_ref, be_ref, Wb_ref, bb_ref):
    t = jnp.dot(hin, Wa_ref[...], preferred_element_type=jnp.float32)
    t = t + ba_ref[...]
    mu = jnp.mean(t, axis=0, keepdims=True)
    var = jnp.mean(jnp.square(t - mu), axis=0, keepdims=True)
    tn = g_ref[...] * (t - mu) * jax.lax.rsqrt(var + 1e-5) + be_ref[...]
    r = jnp.maximum(tn, 0.0)
    h = jnp.dot(r, Wb_ref[...], preferred_element_type=jnp.float32)
    return jnp.maximum(h + bb_ref[...], 0.0)


def _mlp1_kernel(x_ref, p_ref, Wa_ref, ba_ref, g_ref, be_ref, Wb_ref, bb_ref,
                 out_ref):
    hin = x_ref[...] + p_ref[0, :N] + p_ref[1, :N]
    out_ref[...] = _gin_mlp_body(hin, Wa_ref, ba_ref, g_ref, be_ref, Wb_ref,
                                 bb_ref)


def _gin_mlp1(xin, partials, Wa, ba, g, be, Wb, bb):
    return pl.pallas_call(
        _mlp1_kernel,
        out_shape=jax.ShapeDtypeStruct((N, D), jnp.float32),
    )(xin, partials, Wa, ba.reshape(1, D), g.reshape(1, D), be.reshape(1, D),
      Wb, bb.reshape(1, D))


def _mlp2_pool_kernel(h1_ref, p_ref, Wa_ref, ba_ref, g_ref, be_ref, Wb_ref,
                      bb_ref, b_ref, W1_ref, b1_ref, W2_ref, b2_ref, out_ref):
    hin = h1_ref[...] + p_ref[0, :N] + p_ref[1, :N]
    h2 = _gin_mlp_body(hin, Wa_ref, ba_ref, g_ref, be_ref, Wb_ref, bb_ref)
    gids = jax.lax.broadcasted_iota(jnp.int32, (G, N), 0)
    onehot = (gids == b_ref[...]).astype(jnp.float32)
    hg1 = jnp.dot(onehot, h1_ref[...], preferred_element_type=jnp.float32)
    hg2 = jnp.dot(onehot, h2, preferred_element_type=jnp.float32)
    hg = jnp.concatenate([hg1, hg2], axis=1)
    y = jnp.dot(hg, W1_ref[...], preferred_element_type=jnp.float32)
    y = jnp.maximum(y + b1_ref[...], 0.0)
    out_ref[...] = jnp.dot(y, W2_ref[...],
                           preferred_element_type=jnp.float32) + b2_ref[...]


def _mlp2_pool(h1, partials, Wa, ba, g, be, Wb, bb, batch,
               lin1_W, lin1_b, lin2_W, lin2_b):
    return pl.pallas_call(
        _mlp2_pool_kernel,
        out_shape=jax.ShapeDtypeStruct((G, OUT), jnp.float32),
    )(h1, partials, Wa, ba.reshape(1, D), g.reshape(1, D), be.reshape(1, D),
      Wb, bb.reshape(1, D), batch.reshape(1, N), lin1_W,
      lin1_b.reshape(1, D), lin2_W, lin2_b.reshape(1, OUT))


def kernel(x, edge_index, batch, emb, W1a, b1a, g1, be1, W1b, b1b,
           W2a, b2a, g2, be2, W2b, b2b, lin1_W, lin1_b, lin2_W, lin2_b):
    emb_pad = jnp.zeros((VPAD, D), jnp.float32).at[:emb.shape[0]].set(emb)
    eidx = edge_index.reshape(2 * E)

    feats = _embed(x.astype(jnp.int32), emb_pad)

    p1 = _edge_agg_partials(feats, eidx)
    h1 = _gin_mlp1(feats, p1, W1a, b1a, g1, be1, W1b, b1b)

    p2 = _edge_agg_partials(h1, eidx)
    out = _mlp2_pool(h1, p2, W2a, b2a, g2, be2, W2b, b2b, batch,
                     lin1_W, lin1_b, lin2_W, lin2_b)
    return (out, feats)
